# baseline (speedup 1.0000x reference)
import functools

import jax
import jax.numpy as jnp
from jax import lax
from jax.experimental import pallas as pl
from jax.experimental.pallas import tpu as pltpu
from jax.experimental.pallas import tpu_sc as plsc

_B = 1024
_N = 50000
_NV, _NW, _NA, _NF = 12800, 2048, 4096, 1024
_J = _NV + _NW + _NA + _NF
_L = 16
_NWORKERS = 32
_RPW = _B // _NWORKERS
_ZCHUNK = 256
_ROWBUF = ((_N + _ZCHUNK - 1) // _ZCHUNK) * _ZCHUNK
_UNROLL = 8
_SEG_GROUPS = (_NV // _L, _NW // _L, _NA // _L, _NF // _L)


def _sc_body(id_hbm, w_hbm, b_hbm, x_hbm, out_hbm, id_v, w_v, b_v, x_v, row_v):
    wid = lax.axis_index("s") * 2 + lax.axis_index("c")
    pltpu.sync_copy(id_hbm, id_v)
    pltpu.sync_copy(w_hbm, w_v)
    pltpu.sync_copy(b_hbm, b_v)
    pltpu.sync_copy(x_hbm.at[pl.ds(wid * (_RPW * 4 * _L), _RPW * 4 * _L)], x_v)

    zero16 = jnp.zeros((_L,), jnp.float32)

    def row_body(r, carry):
        def zbody(i, c):
            base = i * _ZCHUNK
            for k in range(_ZCHUNK // _L):
                row_v[pl.ds(base + k * _L, _L)] = zero16
            return c

        lax.fori_loop(0, _ROWBUF // _ZCHUNK, zbody, 0)

        seg_base = 0
        for m, ngroups in enumerate(_SEG_GROUPS):
            xvec = x_v[pl.ds(r * (4 * _L) + m * _L, _L)]

            def sbody(g, c, sb=seg_base, xv=xvec):
                for k in range(_UNROLL):
                    off = sb + (g * _UNROLL + k) * _L
                    idv = id_v[pl.ds(off, _L)]
                    wv = w_v[pl.ds(off, _L)]
                    bv = b_v[pl.ds(off, _L)]
                    val = jnp.maximum(wv * xv + bv, 0.0)
                    plsc.addupdate_scatter(row_v, [idv], val)
                return c

            lax.fori_loop(0, ngroups // _UNROLL, sbody, 0)
            seg_base += ngroups * _L

        pltpu.sync_copy(row_v.at[pl.ds(0, _N)],
                        out_hbm.at[pl.ds((wid * _RPW + r) * _N, _N)])
        return carry

    lax.fori_loop(0, _RPW, row_body, 0)


_scatter_call = functools.partial(
    pl.kernel,
    mesh=plsc.VectorSubcoreMesh(core_axis_name="c", subcore_axis_name="s"),
    out_type=jax.ShapeDtypeStruct((_B * _N,), jnp.float32),
    compiler_params=pltpu.CompilerParams(needs_layout_passes=False),
    scratch_types=[
        pltpu.VMEM((_J,), jnp.int32),
        pltpu.VMEM((_J,), jnp.float32),
        pltpu.VMEM((_J,), jnp.float32),
        pltpu.VMEM((_RPW * 4 * _L,), jnp.float32),
        pltpu.VMEM((_ROWBUF,), jnp.float32),
    ],
)(_sc_body)


def kernel(vision, wind_gravity, an, fake_target,
           W_vision, b_vision, W_wind, b_wind, W_an, b_an, W_fake, b_fake,
           id_vision, id_wind, id_an, id_fake):
    w_m = jnp.concatenate(
        [W_vision[:, 0], W_wind[:, 0], W_an[:, 0], W_fake[:, 0]])
    b_m = jnp.concatenate([b_vision, b_wind, b_an, b_fake])
    id_m = jnp.concatenate([id_vision, id_wind, id_an, id_fake])
    x = jnp.concatenate([vision, wind_gravity, an, fake_target], axis=1)
    x_splat = jnp.broadcast_to(x[:, :, None], (_B, 4, _L)).reshape(-1)
    return _scatter_call(id_m, w_m, b_m, x_splat).reshape(_B, _N)


# speedup vs baseline: 1.2293x; 1.2293x over previous
import functools

import jax
import jax.numpy as jnp
from jax import lax
from jax.experimental import pallas as pl
from jax.experimental.pallas import tpu as pltpu
from jax.experimental.pallas import tpu_sc as plsc

_B = 1024
_N = 50000
_NV, _NW, _NA, _NF = 12800, 2048, 4096, 1024
_J = _NV + _NW + _NA + _NF
_L = 16
_NWORKERS = 32
_RPW = _B // _NWORKERS
_H0 = 25088
_H1 = _N - _H0
_HB = _H0
_ZCHUNK = 256
_UNROLL = 8
_SEG_GROUPS = (_NV // _L, _NW // _L, _NA // _L, _NF // _L)


def _sc_body(id_hbm, w_hbm, b_hbm, x_hbm, out_hbm,
             id_v, w_v, b_v, x_v, bh0, bh1, os0, os1):
    wid = lax.axis_index("s") * 2 + lax.axis_index("c")
    pltpu.sync_copy(id_hbm, id_v)
    pltpu.sync_copy(w_hbm, w_v)
    pltpu.sync_copy(b_hbm, b_v)
    pltpu.sync_copy(x_hbm.at[pl.ds(wid * (_RPW * 4 * _L), _RPW * 4 * _L)], x_v)

    zero16 = jnp.zeros((_L,), jnp.float32)

    def row_body(r, carry):
        row = wid * _RPW + r

        @pl.when(r > 0)
        def _():
            pltpu.make_async_copy(bh0.at[pl.ds(0, _H0)],
                                  out_hbm.at[pl.ds(0, _H0)], os0).wait()
            pltpu.make_async_copy(bh1.at[pl.ds(0, _H1)],
                                  out_hbm.at[pl.ds(0, _H1)], os1).wait()

        for buf in (bh0, bh1):
            def zbody(z, c, bf=buf):
                base = z * _ZCHUNK
                for k in range(_ZCHUNK // _L):
                    bf[pl.ds(base + k * _L, _L)] = zero16
                return c

            lax.fori_loop(0, _HB // _ZCHUNK, zbody, 0)

        seg_base = 0
        for m, ngroups in enumerate(_SEG_GROUPS):
            xv = x_v[pl.ds(r * (4 * _L) + m * _L, _L)]
            nsets = ngroups // _UNROLL

            def load_compute(gi, xvec=xv, sb=seg_base):
                base0 = sb + gi * (_UNROLL * _L)
                ids, vals = [], []
                for k in range(_UNROLL):
                    idv = id_v[pl.ds(base0 + k * _L, _L)]
                    wv = w_v[pl.ds(base0 + k * _L, _L)]
                    bv = b_v[pl.ds(base0 + k * _L, _L)]
                    ids.append(idv)
                    vals.append(jnp.maximum(wv * xvec + bv, 0.0))
                return tuple(ids), tuple(vals)

            def sbody(g, carry2, ns=nsets):
                idvs, vals = carry2
                for k in range(_UNROLL):
                    idv = idvs[k]
                    plsc.addupdate_scatter(bh0, [idv], vals[k],
                                           mask=idv < _H0)
                    plsc.addupdate_scatter(bh1, [idv - _H0], vals[k],
                                           mask=idv >= _H0)
                return load_compute(jnp.minimum(g + 1, ns - 1))

            lax.fori_loop(0, nsets, sbody, load_compute(0))
            seg_base += ngroups * _L

        pltpu.async_copy(bh0.at[pl.ds(0, _H0)],
                         out_hbm.at[pl.ds(row * _N, _H0)], os0)
        pltpu.async_copy(bh1.at[pl.ds(0, _H1)],
                         out_hbm.at[pl.ds(row * _N + _H0, _H1)], os1)
        return carry

    lax.fori_loop(0, _RPW, row_body, 0)
    pltpu.make_async_copy(bh0.at[pl.ds(0, _H0)],
                          out_hbm.at[pl.ds(0, _H0)], os0).wait()
    pltpu.make_async_copy(bh1.at[pl.ds(0, _H1)],
                          out_hbm.at[pl.ds(0, _H1)], os1).wait()


_scatter_call = functools.partial(
    pl.kernel,
    mesh=plsc.VectorSubcoreMesh(core_axis_name="c", subcore_axis_name="s"),
    out_type=jax.ShapeDtypeStruct((_B * _N,), jnp.float32),
    compiler_params=pltpu.CompilerParams(needs_layout_passes=False),
    scratch_types=[
        pltpu.VMEM((_J,), jnp.int32),
        pltpu.VMEM((_J,), jnp.float32),
        pltpu.VMEM((_J,), jnp.float32),
        pltpu.VMEM((_RPW * 4 * _L,), jnp.float32),
        pltpu.VMEM((_HB,), jnp.float32),
        pltpu.VMEM((_HB,), jnp.float32),
        pltpu.SemaphoreType.DMA,
        pltpu.SemaphoreType.DMA,
    ],
)(_sc_body)


def kernel(vision, wind_gravity, an, fake_target,
           W_vision, b_vision, W_wind, b_wind, W_an, b_an, W_fake, b_fake,
           id_vision, id_wind, id_an, id_fake):
    w_m = jnp.concatenate(
        [W_vision[:, 0], W_wind[:, 0], W_an[:, 0], W_fake[:, 0]])
    b_m = jnp.concatenate([b_vision, b_wind, b_an, b_fake])
    id_m = jnp.concatenate([id_vision, id_wind, id_an, id_fake])
    x = jnp.concatenate([vision, wind_gravity, an, fake_target], axis=1)
    x_splat = jnp.broadcast_to(x[:, :, None], (_B, 4, _L)).reshape(-1)
    return _scatter_call(id_m, w_m, b_m, x_splat).reshape(_B, _N)
